# baseline (speedup 1.0000x reference)
import functools

import jax
import jax.numpy as jnp
from jax import lax
from jax.experimental import pallas as pl
from jax.experimental.pallas import tpu as pltpu
from jax.experimental.pallas import tpu_sc as plsc

B, L, V, D, H, NB = 4096, 200, 1000000, 64, 256, 4
NC, NS = 2, 16
NW = NC * NS
ROWS_W = B // NW
CHUNK = 100
CPR = L // CHUNK
NCH = ROWS_W * CPR
NLANE = 16
NVEC = D // NLANE

TCOLS = V // 128
VTAIL = V - TCOLS * 128
BASEC = TCOLS // NW
EXTRA = TCOLS - BASEC * NW
SLOTS = BASEC + 1


def _transpose_body(tblT_hbm, out_hbm, tilebuf, rowbuf, si0, si1, so0, so1):
    wid = lax.axis_index("s") * NC + lax.axis_index("c")
    ncw = jnp.where(wid < EXTRA, BASEC + 1, BASEC)
    c0 = jnp.where(
        wid < EXTRA, wid * (BASEC + 1), EXTRA * (BASEC + 1) + (wid - EXTRA) * BASEC
    )
    sin = (si0, si1)
    sout = (so0, so1)

    iota = lax.iota(jnp.int32, NLANE)
    dds = [iota + 16 * m for m in range(4)]

    def start_in(c, p):
        pltpu.async_copy(
            tblT_hbm.at[pl.ds(0, 64), pl.ds(c * 128, 128)],
            tilebuf.at[p, :, pl.ds(0, 128)],
            sin[p],
        )

    def wait_in(p):
        pltpu.make_async_copy(
            tblT_hbm.at[pl.ds(0, 64), pl.ds(0, 128)],
            tilebuf.at[p, :, pl.ds(0, 128)],
            sin[p],
        ).wait()

    def start_out(c, p):
        pltpu.async_copy(rowbuf.at[p], out_hbm.at[pl.ds(c * 8192, 8192)], sout[p])

    def wait_out(p):
        pltpu.make_async_copy(
            rowbuf.at[p], out_hbm.at[pl.ds(0, 8192)], sout[p]
        ).wait()

    def transpose(p):
        @plsc.parallel_loop(0, 64, step=1, unroll=8)
        def _(j):
            for half in range(2):
                v = 2 * j + half
                vsplat = jnp.full((NLANE,), 0, jnp.int32) + v
                for m in range(4):
                    vec = plsc.load_gather(tilebuf.at[p], [dds[m], vsplat])
                    rowbuf[p, pl.ds(j * 128 + half * 64 + m * 16, NLANE)] = vec

    for p in range(2):
        start_in(c0 + p, p)

    def slot(s, p, first):
        valid = s < ncw
        c = c0 + s

        @pl.when(valid)
        def _():
            wait_in(p)

        if not first:
            wait_out(p)

        @pl.when(valid)
        def _():
            transpose(p)
            start_out(c, p)

        @pl.when(s + 2 < ncw)
        def _():
            start_in(c + 2, p)

    slot(0, 0, True)
    slot(1, 1, True)

    def pair(g, _):
        s = 2 * g
        slot(s, 0, False)
        slot(s + 1, 1, False)
        return 0

    lax.fori_loop(1, (SLOTS + 1) // 2, pair, 0)
    @pl.when(ncw == BASEC + 1)
    def _():
        wait_out(0)


def _sc_transpose(tblT):
    mesh = plsc.VectorSubcoreMesh(core_axis_name="c", subcore_axis_name="s")
    return pl.kernel(
        _transpose_body,
        out_type=jax.ShapeDtypeStruct((V * D,), jnp.float32),
        mesh=mesh,
        scratch_types=[
            pltpu.VMEM((2, 64, 145), jnp.float32),
            pltpu.VMEM((2, 8192), jnp.float32),
            pltpu.SemaphoreType.DMA,
            pltpu.SemaphoreType.DMA,
            pltpu.SemaphoreType.DMA,
            pltpu.SemaphoreType.DMA,
        ],
        compiler_params=pltpu.CompilerParams(
            use_tc_tiling_on_sc=True, needs_layout_passes=False
        ),
    )(tblT)


def _sc_gather_mean_body(ids_hbm, table_hbm, out_hbm, idx_v, rows_v, acc_v, sem0, sem1):
    wid = lax.axis_index("s") * NC + lax.axis_index("c")
    pltpu.sync_copy(ids_hbm.at[wid], idx_v)
    sems = (sem0, sem1)

    def fetch(r, p):
        for c in range(CPR):
            pltpu.async_copy(
                table_hbm.at[idx_v.at[CPR * r + c]], rows_v.at[p, c], sems[p]
            )

    for p in range(2):
        fetch(p, p)

    def pair_body(g, _):
        for p in range(2):
            r = 2 * g + p
            for c in range(CPR):
                pltpu.make_async_copy(
                    table_hbm.at[pl.ds(0, CHUNK)], rows_v.at[p, c], sems[p]
                ).wait()

            def acc_body(l, accs):
                accs = list(accs)
                for u in range(2):
                    for c in range(CPR):
                        for k in range(NVEC):
                            accs[c * NVEC + k] = accs[c * NVEC + k] + rows_v[
                                p, c, 2 * l + u, pl.ds(k * NLANE, NLANE)
                            ]
                return tuple(accs)

            accs = lax.fori_loop(
                0,
                CHUNK // 2,
                acc_body,
                tuple(jnp.zeros((NLANE,), jnp.float32) for _ in range(CPR * NVEC)),
            )
            scale = jnp.float32(1.0 / L)
            for k in range(NVEC):
                tot = accs[k]
                for c in range(1, CPR):
                    tot = tot + accs[c * NVEC + k]
                acc_v[r, pl.ds(k * NLANE, NLANE)] = tot * scale

            nr = r + 2

            @pl.when(nr < ROWS_W)
            def _():
                fetch(nr, p)

        return 0

    lax.fori_loop(0, ROWS_W // 2, pair_body, 0)
    pltpu.sync_copy(acc_v, out_hbm.at[pl.ds(wid * ROWS_W, ROWS_W)])


def _sc_gather_mean(ids3, table):
    mesh = plsc.VectorSubcoreMesh(core_axis_name="c", subcore_axis_name="s")
    return pl.kernel(
        _sc_gather_mean_body,
        out_type=jax.ShapeDtypeStruct((B, D), jnp.float32),
        mesh=mesh,
        scratch_types=[
            pltpu.VMEM((NCH, CHUNK), jnp.int32),
            pltpu.VMEM((2, CPR, CHUNK, D), jnp.float32),
            pltpu.VMEM((ROWS_W, D), jnp.float32),
            pltpu.SemaphoreType.DMA,
            pltpu.SemaphoreType.DMA,
        ],
        compiler_params=pltpu.CompilerParams(use_tc_tiling_on_sc=False),
    )(ids3, table)


def _mlp_body(cur_ref, ctx_ref, w1a_ref, w1b_ref, b1_ref, w2_ref, b2_ref, out_ref):
    cur = cur_ref[...]
    ctx = ctx_ref[...]
    for i in range(NB):
        h = (
            jnp.dot(cur, w1a_ref[i], preferred_element_type=jnp.float32)
            + jnp.dot(ctx, w1b_ref[i], preferred_element_type=jnp.float32)
            + b1_ref[i][None, :]
        )
        h = jnp.maximum(h, 0.0)
        cur = cur + jnp.dot(h, w2_ref[i], preferred_element_type=jnp.float32) + b2_ref[i][None, :]
    out_ref[...] = cur


def _tc_mlp(cur0, ctx, W1, b1, W2, b2):
    w1a = W1[:, :D, :]
    w1b = W1[:, D:, :]
    bm = 512
    grid = B // bm
    return pl.pallas_call(
        _mlp_body,
        grid=(grid,),
        in_specs=[
            pl.BlockSpec((bm, D), lambda i: (i, 0)),
            pl.BlockSpec((bm, D), lambda i: (i, 0)),
            pl.BlockSpec((NB, D, H), lambda i: (0, 0, 0)),
            pl.BlockSpec((NB, D, H), lambda i: (0, 0, 0)),
            pl.BlockSpec((NB, H), lambda i: (0, 0)),
            pl.BlockSpec((NB, H, D), lambda i: (0, 0, 0)),
            pl.BlockSpec((NB, D), lambda i: (0, 0)),
        ],
        out_specs=pl.BlockSpec((bm, D), lambda i: (i, 0)),
        out_shape=jax.ShapeDtypeStruct((B, D), jnp.float32),
    )(cur0, ctx, w1a, w1b, b1, W2, b2)


def kernel(initial_noisy_embedding, context_ids, embedding_table, W1, b1, W2, b2):
    ids3 = context_ids.astype(jnp.int32).reshape(NW, NCH, CHUNK)
    scratch = _sc_transpose(embedding_table.T)
    tail = lax.slice(embedding_table, (TCOLS * 128, 0), (V, D)).reshape(-1)
    scratch = lax.dynamic_update_slice(scratch, tail, (TCOLS * 128 * D,))
    table_lin = scratch.reshape(V, D)
    ctx = _sc_gather_mean(ids3, table_lin)
    return _tc_mlp(initial_noisy_embedding, ctx, W1, b1, W2, b2)


# speedup vs baseline: 1.0162x; 1.0162x over previous
import functools

import jax
import jax.numpy as jnp
from jax import lax
from jax.experimental import pallas as pl
from jax.experimental.pallas import tpu as pltpu
from jax.experimental.pallas import tpu_sc as plsc

B, L, V, D, H, NB = 4096, 200, 1000000, 64, 256, 4
NC, NS = 2, 16
NW = NC * NS
ROWS_W = B // NW
CHUNK = 100
CPR = L // CHUNK
NCH = ROWS_W * CPR
NLANE = 16
NVEC = D // NLANE

TCOLS = V // 128
VTAIL = V - TCOLS * 128
BASEC = TCOLS // NW
EXTRA = TCOLS - BASEC * NW
SLOTS = BASEC + 1


def _transpose_body(tblT_hbm, out_hbm, tb0, tb1, rb0, rb1, si0, si1, so0, so1):
    wid = lax.axis_index("s") * NC + lax.axis_index("c")
    ncw = jnp.where(wid < EXTRA, BASEC + 1, BASEC)
    c0 = jnp.where(
        wid < EXTRA, wid * (BASEC + 1), EXTRA * (BASEC + 1) + (wid - EXTRA) * BASEC
    )
    sin = (si0, si1)
    sout = (so0, so1)
    tbs = (tb0, tb1)
    rbs = (rb0, rb1)

    iota = lax.iota(jnp.int32, NLANE)
    dds = [iota + 16 * m for m in range(4)]
    vbs = [(iota + 16 * g) * D for g in range(8)]

    def start_in(c, p):
        pltpu.async_copy(
            tblT_hbm.at[pl.ds(0, 64), pl.ds(c * 128, 128)],
            tbs[p],
            sin[p],
        )

    def wait_in(p):
        pltpu.make_async_copy(
            tblT_hbm.at[pl.ds(0, 64), pl.ds(0, 128)],
            tbs[p],
            sin[p],
        ).wait()

    def start_out(c, p):
        pltpu.async_copy(rbs[p], out_hbm.at[pl.ds(c * 8192, 8192)], sout[p])

    def wait_out(p):
        pltpu.make_async_copy(
            rbs[p], out_hbm.at[pl.ds(0, 8192)], sout[p]
        ).wait()

    def transpose(p):
        @plsc.parallel_loop(0, 64, step=1, unroll=8)
        def _(d):
            dsplat = jnp.full((NLANE,), 0, jnp.int32) + d
            for g in range(8):
                vec = plsc.load_gather(tbs[p], [dsplat, iota + g * 16])
                plsc.store_scatter(rbs[p], [vbs[g] + d], vec)

    for p in range(2):
        start_in(c0 + p, p)

    def slot(s, p, first):
        valid = s < ncw
        c = c0 + s

        @pl.when(valid)
        def _():
            wait_in(p)

        if not first:
            wait_out(p)

        @pl.when(valid)
        def _():
            transpose(p)
            start_out(c, p)

        @pl.when(s + 2 < ncw)
        def _():
            start_in(c + 2, p)

    slot(0, 0, True)
    slot(1, 1, True)

    def pair(g, _):
        s = 2 * g
        slot(s, 0, False)
        slot(s + 1, 1, False)
        return 0

    lax.fori_loop(1, (SLOTS + 1) // 2, pair, 0)
    @pl.when(ncw == BASEC + 1)
    def _():
        wait_out(0)


def _sc_transpose(tblT):
    mesh = plsc.VectorSubcoreMesh(core_axis_name="c", subcore_axis_name="s")
    return pl.kernel(
        _transpose_body,
        out_type=jax.ShapeDtypeStruct((V * D,), jnp.float32),
        mesh=mesh,
        scratch_types=[
            pltpu.VMEM((64, 128), jnp.float32),
            pltpu.VMEM((64, 128), jnp.float32),
            pltpu.VMEM((8192,), jnp.float32),
            pltpu.VMEM((8192,), jnp.float32),
            pltpu.SemaphoreType.DMA,
            pltpu.SemaphoreType.DMA,
            pltpu.SemaphoreType.DMA,
            pltpu.SemaphoreType.DMA,
        ],
        compiler_params=pltpu.CompilerParams(
            use_tc_tiling_on_sc=True, needs_layout_passes=False
        ),
    )(tblT)


def _sc_gather_mean_body(ids_hbm, table_hbm, out_hbm, idx_v, rows_v, acc_v, sem0, sem1):
    wid = lax.axis_index("s") * NC + lax.axis_index("c")
    pltpu.sync_copy(ids_hbm.at[wid], idx_v)
    sems = (sem0, sem1)

    def fetch(r, p):
        for c in range(CPR):
            pltpu.async_copy(
                table_hbm.at[idx_v.at[CPR * r + c]], rows_v.at[p, c], sems[p]
            )

    for p in range(2):
        fetch(p, p)

    def pair_body(g, _):
        for p in range(2):
            r = 2 * g + p
            for c in range(CPR):
                pltpu.make_async_copy(
                    table_hbm.at[pl.ds(0, CHUNK)], rows_v.at[p, c], sems[p]
                ).wait()

            def acc_body(l, accs):
                accs = list(accs)
                for u in range(2):
                    for c in range(CPR):
                        for k in range(NVEC):
                            accs[c * NVEC + k] = accs[c * NVEC + k] + rows_v[
                                p, c, 2 * l + u, pl.ds(k * NLANE, NLANE)
                            ]
                return tuple(accs)

            accs = lax.fori_loop(
                0,
                CHUNK // 2,
                acc_body,
                tuple(jnp.zeros((NLANE,), jnp.float32) for _ in range(CPR * NVEC)),
            )
            scale = jnp.float32(1.0 / L)
            for k in range(NVEC):
                tot = accs[k]
                for c in range(1, CPR):
                    tot = tot + accs[c * NVEC + k]
                acc_v[r, pl.ds(k * NLANE, NLANE)] = tot * scale

            nr = r + 2

            @pl.when(nr < ROWS_W)
            def _():
                fetch(nr, p)

        return 0

    lax.fori_loop(0, ROWS_W // 2, pair_body, 0)
    pltpu.sync_copy(acc_v, out_hbm.at[pl.ds(wid * ROWS_W, ROWS_W)])


def _sc_gather_mean(ids3, table):
    mesh = plsc.VectorSubcoreMesh(core_axis_name="c", subcore_axis_name="s")
    return pl.kernel(
        _sc_gather_mean_body,
        out_type=jax.ShapeDtypeStruct((B, D), jnp.float32),
        mesh=mesh,
        scratch_types=[
            pltpu.VMEM((NCH, CHUNK), jnp.int32),
            pltpu.VMEM((2, CPR, CHUNK, D), jnp.float32),
            pltpu.VMEM((ROWS_W, D), jnp.float32),
            pltpu.SemaphoreType.DMA,
            pltpu.SemaphoreType.DMA,
        ],
        compiler_params=pltpu.CompilerParams(use_tc_tiling_on_sc=False),
    )(ids3, table)


def _mlp_body(cur_ref, ctx_ref, w1a_ref, w1b_ref, b1_ref, w2_ref, b2_ref, out_ref):
    cur = cur_ref[...]
    ctx = ctx_ref[...]
    for i in range(NB):
        h = (
            jnp.dot(cur, w1a_ref[i], preferred_element_type=jnp.float32)
            + jnp.dot(ctx, w1b_ref[i], preferred_element_type=jnp.float32)
            + b1_ref[i][None, :]
        )
        h = jnp.maximum(h, 0.0)
        cur = cur + jnp.dot(h, w2_ref[i], preferred_element_type=jnp.float32) + b2_ref[i][None, :]
    out_ref[...] = cur


def _tc_mlp(cur0, ctx, W1, b1, W2, b2):
    w1a = W1[:, :D, :]
    w1b = W1[:, D:, :]
    bm = 512
    grid = B // bm
    return pl.pallas_call(
        _mlp_body,
        grid=(grid,),
        in_specs=[
            pl.BlockSpec((bm, D), lambda i: (i, 0)),
            pl.BlockSpec((bm, D), lambda i: (i, 0)),
            pl.BlockSpec((NB, D, H), lambda i: (0, 0, 0)),
            pl.BlockSpec((NB, D, H), lambda i: (0, 0, 0)),
            pl.BlockSpec((NB, H), lambda i: (0, 0)),
            pl.BlockSpec((NB, H, D), lambda i: (0, 0, 0)),
            pl.BlockSpec((NB, D), lambda i: (0, 0)),
        ],
        out_specs=pl.BlockSpec((bm, D), lambda i: (i, 0)),
        out_shape=jax.ShapeDtypeStruct((B, D), jnp.float32),
    )(cur0, ctx, w1a, w1b, b1, W2, b2)


def kernel(initial_noisy_embedding, context_ids, embedding_table, W1, b1, W2, b2):
    ids3 = context_ids.astype(jnp.int32).reshape(NW, NCH, CHUNK)
    scratch = _sc_transpose(embedding_table.T)
    tail = lax.slice(embedding_table, (TCOLS * 128, 0), (V, D)).reshape(-1)
    scratch = lax.dynamic_update_slice(scratch, tail, (TCOLS * 128 * D,))
    table_lin = scratch.reshape(V, D)
    ctx = _sc_gather_mean(ids3, table_lin)
    return _tc_mlp(initial_noisy_embedding, ctx, W1, b1, W2, b2)


# speedup vs baseline: 2.7769x; 2.7328x over previous
import functools

import jax
import jax.numpy as jnp
from jax import lax
from jax.experimental import pallas as pl
from jax.experimental.pallas import tpu as pltpu
from jax.experimental.pallas import tpu_sc as plsc

B, L, V, D, H, NB = 4096, 200, 1000000, 64, 256, 4
NC, NS = 2, 16
NW = NC * NS
ROWS_W = B // NW
CHUNK = 100
CPR = L // CHUNK
NCH = ROWS_W * CPR
NLANE = 16
NVEC = D // NLANE

TCOLS = V // 128
VTAIL = V - TCOLS * 128
BASEC = TCOLS // NW
EXTRA = TCOLS - BASEC * NW
SLOTS = BASEC + 1


def _transpose_body(tblT_hbm, out_hbm, tb0, tb1, rb0, rb1, si0, si1, so0, so1):
    wid = lax.axis_index("s") * NC + lax.axis_index("c")
    ncw = jnp.where(wid < EXTRA, BASEC + 1, BASEC)
    c0 = jnp.where(
        wid < EXTRA, wid * (BASEC + 1), EXTRA * (BASEC + 1) + (wid - EXTRA) * BASEC
    )
    sin = (si0, si1)
    sout = (so0, so1)
    tbs = (tb0, tb1)
    rbs = (rb0, rb1)

    iota = lax.iota(jnp.int32, NLANE)
    dds = [iota + 16 * m for m in range(4)]
    vbs = [(iota + 16 * g) * D for g in range(8)]

    def start_in(c, p):
        pltpu.async_copy(
            tblT_hbm.at[pl.ds(0, 64), pl.ds(c * 128, 128)],
            tbs[p],
            sin[p],
        )

    def wait_in(p):
        pltpu.make_async_copy(
            tblT_hbm.at[pl.ds(0, 64), pl.ds(0, 128)],
            tbs[p],
            sin[p],
        ).wait()

    def start_out(c, p):
        pltpu.async_copy(rbs[p], out_hbm.at[pl.ds(c * 8192, 8192)], sout[p])

    def wait_out(p):
        pltpu.make_async_copy(
            rbs[p], out_hbm.at[pl.ds(0, 8192)], sout[p]
        ).wait()

    def transpose(p):
        @plsc.parallel_loop(0, 64, step=1, unroll=8)
        def _(d):
            dsplat = jnp.full((NLANE,), 0, jnp.int32) + d
            for g in range(8):
                vec = plsc.load_gather(tbs[p], [dsplat, iota + g * 16])
                plsc.store_scatter(rbs[p], [vbs[g] + d], vec)

    for p in range(2):
        start_in(c0 + p, p)

    def slot(s, p, first):
        valid = s < ncw
        c = c0 + s

        @pl.when(valid)
        def _():
            wait_in(p)

        if not first:
            wait_out(p)

        @pl.when(valid)
        def _():
            transpose(p)
            start_out(c, p)

        @pl.when(s + 2 < ncw)
        def _():
            start_in(c + 2, p)

    slot(0, 0, True)
    slot(1, 1, True)

    def pair(g, _):
        s = 2 * g
        slot(s, 0, False)
        slot(s + 1, 1, False)
        return 0

    lax.fori_loop(1, (SLOTS + 1) // 2, pair, 0)
    @pl.when(ncw == BASEC + 1)
    def _():
        wait_out(0)


def _sc_transpose(tblT):
    mesh = plsc.VectorSubcoreMesh(core_axis_name="c", subcore_axis_name="s")
    return pl.kernel(
        _transpose_body,
        out_type=jax.ShapeDtypeStruct((V * D,), jnp.float32),
        mesh=mesh,
        scratch_types=[
            pltpu.VMEM((64, 128), jnp.float32),
            pltpu.VMEM((64, 128), jnp.float32),
            pltpu.VMEM((8192,), jnp.float32),
            pltpu.VMEM((8192,), jnp.float32),
            pltpu.SemaphoreType.DMA,
            pltpu.SemaphoreType.DMA,
            pltpu.SemaphoreType.DMA,
            pltpu.SemaphoreType.DMA,
        ],
        compiler_params=pltpu.CompilerParams(
            use_tc_tiling_on_sc=True, needs_layout_passes=False
        ),
    )(tblT)


TBLK = 12800
TGRID = (V + TBLK - 1) // TBLK


def _tct_body(in_ref, out_ref):
    iot_r = lax.broadcasted_iota(jnp.int32, (128, 256), 0)
    iot_c = lax.broadcasted_iota(jnp.int32, (128, 256), 1)
    ee = (iot_c == 2 * iot_r).astype(jnp.float32)
    eo = (iot_c == 2 * iot_r + 1).astype(jnp.float32)
    dn = (((1,), (1,)), ((), ()))
    for b in range(TBLK // 256):
        xb = in_ref[:, pl.ds(b * 256, 256)]
        te = lax.dot_general(ee, xb, dn, preferred_element_type=jnp.float32)
        to = lax.dot_general(eo, xb, dn, preferred_element_type=jnp.float32)
        out_ref[pl.ds(b * 128, 128), pl.ds(0, D)] = te
        out_ref[pl.ds(b * 128, 128), pl.ds(D, D)] = to


def _tc_transpose(tblT):
    return pl.pallas_call(
        _tct_body,
        grid=(TGRID,),
        in_specs=[pl.BlockSpec((D, TBLK), lambda i: (0, i))],
        out_specs=pl.BlockSpec((TBLK // 2, 128), lambda i: (i, 0)),
        out_shape=jax.ShapeDtypeStruct((V // 2, 128), jnp.float32),
    )(tblT)


def _sc_gather_mean_body(ids_hbm, table_hbm, out_hbm, idx_v, rows_v, acc_v, sem0, sem1):
    wid = lax.axis_index("s") * NC + lax.axis_index("c")
    pltpu.sync_copy(ids_hbm.at[wid], idx_v)
    sems = (sem0, sem1)

    def fetch(r, p):
        for c in range(CPR):
            pltpu.async_copy(
                table_hbm.at[idx_v.at[CPR * r + c]], rows_v.at[p, c], sems[p]
            )

    for p in range(2):
        fetch(p, p)

    def pair_body(g, _):
        for p in range(2):
            r = 2 * g + p
            for c in range(CPR):
                pltpu.make_async_copy(
                    table_hbm.at[pl.ds(0, CHUNK)], rows_v.at[p, c], sems[p]
                ).wait()

            def acc_body(l, accs):
                accs = list(accs)
                for u in range(2):
                    for c in range(CPR):
                        for k in range(NVEC):
                            accs[c * NVEC + k] = accs[c * NVEC + k] + rows_v[
                                p, c, 2 * l + u, pl.ds(k * NLANE, NLANE)
                            ]
                return tuple(accs)

            accs = lax.fori_loop(
                0,
                CHUNK // 2,
                acc_body,
                tuple(jnp.zeros((NLANE,), jnp.float32) for _ in range(CPR * NVEC)),
            )
            scale = jnp.float32(1.0 / L)
            for k in range(NVEC):
                tot = accs[k]
                for c in range(1, CPR):
                    tot = tot + accs[c * NVEC + k]
                acc_v[r, pl.ds(k * NLANE, NLANE)] = tot * scale

            nr = r + 2

            @pl.when(nr < ROWS_W)
            def _():
                fetch(nr, p)

        return 0

    lax.fori_loop(0, ROWS_W // 2, pair_body, 0)
    pltpu.sync_copy(acc_v, out_hbm.at[pl.ds(wid * ROWS_W, ROWS_W)])


def _sc_gather_mean(ids3, table):
    mesh = plsc.VectorSubcoreMesh(core_axis_name="c", subcore_axis_name="s")
    return pl.kernel(
        _sc_gather_mean_body,
        out_type=jax.ShapeDtypeStruct((B, D), jnp.float32),
        mesh=mesh,
        scratch_types=[
            pltpu.VMEM((NCH, CHUNK), jnp.int32),
            pltpu.VMEM((2, CPR, CHUNK, D), jnp.float32),
            pltpu.VMEM((ROWS_W, D), jnp.float32),
            pltpu.SemaphoreType.DMA,
            pltpu.SemaphoreType.DMA,
        ],
        compiler_params=pltpu.CompilerParams(use_tc_tiling_on_sc=False),
    )(ids3, table)


def _mlp_body(cur_ref, ctx_ref, w1a_ref, w1b_ref, b1_ref, w2_ref, b2_ref, out_ref):
    cur = cur_ref[...]
    ctx = ctx_ref[...]
    for i in range(NB):
        h = (
            jnp.dot(cur, w1a_ref[i], preferred_element_type=jnp.float32)
            + jnp.dot(ctx, w1b_ref[i], preferred_element_type=jnp.float32)
            + b1_ref[i][None, :]
        )
        h = jnp.maximum(h, 0.0)
        cur = cur + jnp.dot(h, w2_ref[i], preferred_element_type=jnp.float32) + b2_ref[i][None, :]
    out_ref[...] = cur


def _tc_mlp(cur0, ctx, W1, b1, W2, b2):
    w1a = W1[:, :D, :]
    w1b = W1[:, D:, :]
    bm = 512
    grid = B // bm
    return pl.pallas_call(
        _mlp_body,
        grid=(grid,),
        in_specs=[
            pl.BlockSpec((bm, D), lambda i: (i, 0)),
            pl.BlockSpec((bm, D), lambda i: (i, 0)),
            pl.BlockSpec((NB, D, H), lambda i: (0, 0, 0)),
            pl.BlockSpec((NB, D, H), lambda i: (0, 0, 0)),
            pl.BlockSpec((NB, H), lambda i: (0, 0)),
            pl.BlockSpec((NB, H, D), lambda i: (0, 0, 0)),
            pl.BlockSpec((NB, D), lambda i: (0, 0)),
        ],
        out_specs=pl.BlockSpec((bm, D), lambda i: (i, 0)),
        out_shape=jax.ShapeDtypeStruct((B, D), jnp.float32),
    )(cur0, ctx, w1a, w1b, b1, W2, b2)


def kernel(initial_noisy_embedding, context_ids, embedding_table, W1, b1, W2, b2):
    ids3 = context_ids.astype(jnp.int32).reshape(NW, NCH, CHUNK)
    table_lin = _tc_transpose(embedding_table.T).reshape(V, D)
    ctx = _sc_gather_mean(ids3, table_lin)
    return _tc_mlp(initial_noisy_embedding, ctx, W1, b1, W2, b2)


# speedup vs baseline: 3.1172x; 1.1225x over previous
import functools

import jax
import jax.numpy as jnp
from jax import lax
from jax.experimental import pallas as pl
from jax.experimental.pallas import tpu as pltpu
from jax.experimental.pallas import tpu_sc as plsc

B, L, V, D, H, NB = 4096, 200, 1000000, 64, 256, 4
NC, NS = 2, 16
NW = NC * NS
ROWS_W = B // NW
CHUNK = 100
CPR = L // CHUNK
NCH = ROWS_W * CPR
NLANE = 16
NVEC = D // NLANE


TBLK = 12800
TGRID = (V + TBLK - 1) // TBLK


def _tct_body(in_ref, out_ref):
    iot_r = lax.broadcasted_iota(jnp.int32, (128, 256), 0)
    iot_c = lax.broadcasted_iota(jnp.int32, (128, 256), 1)
    ee = (iot_c == 2 * iot_r).astype(jnp.float32)
    eo = (iot_c == 2 * iot_r + 1).astype(jnp.float32)
    dn = (((1,), (1,)), ((), ()))
    for b in range(TBLK // 256):
        xb = in_ref[:, pl.ds(b * 256, 256)]
        te = lax.dot_general(ee, xb, dn, preferred_element_type=jnp.float32)
        to = lax.dot_general(eo, xb, dn, preferred_element_type=jnp.float32)
        out_ref[pl.ds(b * 128, 128), pl.ds(0, D)] = te
        out_ref[pl.ds(b * 128, 128), pl.ds(D, D)] = to


def _tc_transpose(tblT):
    return pl.pallas_call(
        _tct_body,
        grid=(TGRID,),
        in_specs=[pl.BlockSpec((D, TBLK), lambda i: (0, i))],
        out_specs=pl.BlockSpec((TBLK // 2, 128), lambda i: (i, 0)),
        out_shape=jax.ShapeDtypeStruct((V // 2, 128), jnp.float32),
    )(tblT)


def _sc_gather_mean_body(ids_hbm, table_hbm, out_hbm, idx_v, rows_v, acc_v, sem0, sem1, sem2, sem3):
    wid = lax.axis_index("s") * NC + lax.axis_index("c")
    pltpu.sync_copy(ids_hbm.at[wid], idx_v)
    sems = (sem0, sem1, sem2, sem3)

    def fetch(r, p):
        for c in range(CPR):
            pltpu.async_copy(
                table_hbm.at[idx_v.at[CPR * r + c]], rows_v.at[p, c], sems[p]
            )

    for p in range(4):
        fetch(p, p)

    def pair_body(g, _):
        for p in range(4):
            r = 4 * g + p
            for c in range(CPR):
                pltpu.make_async_copy(
                    table_hbm.at[pl.ds(0, CHUNK)], rows_v.at[p, c], sems[p]
                ).wait()

            def acc_body(l, accs):
                accs = list(accs)
                for u in range(2):
                    for c in range(CPR):
                        for k in range(NVEC):
                            accs[c * NVEC + k] = accs[c * NVEC + k] + rows_v[
                                p, c, 2 * l + u, pl.ds(k * NLANE, NLANE)
                            ]
                return tuple(accs)

            accs = lax.fori_loop(
                0,
                CHUNK // 2,
                acc_body,
                tuple(jnp.zeros((NLANE,), jnp.float32) for _ in range(CPR * NVEC)),
            )
            scale = jnp.float32(1.0 / L)
            for k in range(NVEC):
                tot = accs[k]
                for c in range(1, CPR):
                    tot = tot + accs[c * NVEC + k]
                acc_v[r, pl.ds(k * NLANE, NLANE)] = tot * scale

            nr = r + 4

            @pl.when(nr < ROWS_W)
            def _():
                fetch(nr, p)

        return 0

    lax.fori_loop(0, ROWS_W // 4, pair_body, 0)
    pltpu.sync_copy(acc_v, out_hbm.at[pl.ds(wid * ROWS_W, ROWS_W)])


def _sc_gather_mean(ids3, table):
    mesh = plsc.VectorSubcoreMesh(core_axis_name="c", subcore_axis_name="s")
    return pl.kernel(
        _sc_gather_mean_body,
        out_type=jax.ShapeDtypeStruct((B, D), jnp.float32),
        mesh=mesh,
        scratch_types=[
            pltpu.VMEM((NCH, CHUNK), jnp.int32),
            pltpu.VMEM((4, CPR, CHUNK, D), jnp.float32),
            pltpu.VMEM((ROWS_W, D), jnp.float32),
            pltpu.SemaphoreType.DMA,
            pltpu.SemaphoreType.DMA,
            pltpu.SemaphoreType.DMA,
            pltpu.SemaphoreType.DMA,
        ],
        compiler_params=pltpu.CompilerParams(use_tc_tiling_on_sc=False),
    )(ids3, table)


def _mlp_body(cur_ref, ctx_ref, w1a_ref, w1b_ref, b1_ref, w2_ref, b2_ref, out_ref):
    cur = cur_ref[...]
    ctx = ctx_ref[...]
    for i in range(NB):
        h = (
            jnp.dot(cur, w1a_ref[i], preferred_element_type=jnp.float32)
            + jnp.dot(ctx, w1b_ref[i], preferred_element_type=jnp.float32)
            + b1_ref[i][None, :]
        )
        h = jnp.maximum(h, 0.0)
        cur = cur + jnp.dot(h, w2_ref[i], preferred_element_type=jnp.float32) + b2_ref[i][None, :]
    out_ref[...] = cur


def _tc_mlp(cur0, ctx, W1, b1, W2, b2):
    w1a = W1[:, :D, :]
    w1b = W1[:, D:, :]
    bm = 512
    grid = B // bm
    return pl.pallas_call(
        _mlp_body,
        grid=(grid,),
        in_specs=[
            pl.BlockSpec((bm, D), lambda i: (i, 0)),
            pl.BlockSpec((bm, D), lambda i: (i, 0)),
            pl.BlockSpec((NB, D, H), lambda i: (0, 0, 0)),
            pl.BlockSpec((NB, D, H), lambda i: (0, 0, 0)),
            pl.BlockSpec((NB, H), lambda i: (0, 0)),
            pl.BlockSpec((NB, H, D), lambda i: (0, 0, 0)),
            pl.BlockSpec((NB, D), lambda i: (0, 0)),
        ],
        out_specs=pl.BlockSpec((bm, D), lambda i: (i, 0)),
        out_shape=jax.ShapeDtypeStruct((B, D), jnp.float32),
    )(cur0, ctx, w1a, w1b, b1, W2, b2)


def kernel(initial_noisy_embedding, context_ids, embedding_table, W1, b1, W2, b2):
    ids3 = context_ids.astype(jnp.int32).reshape(NW, NCH, CHUNK)
    table_lin = _tc_transpose(embedding_table.T).reshape(V, D)
    ctx = _sc_gather_mean(ids3, table_lin)
    return _tc_mlp(initial_noisy_embedding, ctx, W1, b1, W2, b2)


# speedup vs baseline: 3.1262x; 1.0029x over previous
import jax
import jax.numpy as jnp
from jax import lax
from jax.experimental import pallas as pl
from jax.experimental.pallas import tpu as pltpu
from jax.experimental.pallas import tpu_sc as plsc

B, L, V, D, H, NB = 4096, 200, 1000000, 64, 256, 4
NC, NS = 2, 16
NW = NC * NS
ROWS_W = B // NW
CHUNK = 100
CPR = L // CHUNK
NCH = ROWS_W * CPR
NLANE = 16
NVEC = D // NLANE


TBLK = 12800
TGRID = (V + TBLK - 1) // TBLK


def _tct_body(in_ref, out_ref):
    iot_r = lax.broadcasted_iota(jnp.int32, (128, 256), 0)
    iot_c = lax.broadcasted_iota(jnp.int32, (128, 256), 1)
    ee = (iot_c == 2 * iot_r).astype(jnp.float32)
    eo = (iot_c == 2 * iot_r + 1).astype(jnp.float32)
    dn = (((1,), (1,)), ((), ()))
    for b in range(TBLK // 256):
        xb = in_ref[:, pl.ds(b * 256, 256)]
        te = lax.dot_general(ee, xb, dn, preferred_element_type=jnp.float32)
        to = lax.dot_general(eo, xb, dn, preferred_element_type=jnp.float32)
        out_ref[pl.ds(b * 128, 128), pl.ds(0, D)] = te
        out_ref[pl.ds(b * 128, 128), pl.ds(D, D)] = to


def _tc_transpose(tblT):
    return pl.pallas_call(
        _tct_body,
        grid=(TGRID,),
        in_specs=[pl.BlockSpec((D, TBLK), lambda i: (0, i))],
        out_specs=pl.BlockSpec((TBLK // 2, 128), lambda i: (i, 0)),
        out_shape=jax.ShapeDtypeStruct((V // 2, 128), jnp.float32),
    )(tblT)


def _sc_gather_mean_body(ids_hbm, table_hbm, out_hbm, idx_v, rows_v, acc_v, sem0, sem1, sem2, sem3):
    wid = lax.axis_index("s") * NC + lax.axis_index("c")
    pltpu.sync_copy(ids_hbm.at[wid], idx_v)
    sems = (sem0, sem1, sem2, sem3)

    def fetch(r, p):
        for c in range(CPR):
            pltpu.async_copy(
                table_hbm.at[idx_v.at[CPR * r + c]], rows_v.at[p, c], sems[p]
            )

    for p in range(4):
        fetch(p, p)

    def pair_body(g, _):
        for p in range(4):
            r = 4 * g + p
            for c in range(CPR):
                pltpu.make_async_copy(
                    table_hbm.at[pl.ds(0, CHUNK)], rows_v.at[p, c], sems[p]
                ).wait()

            def acc_body(l, accs):
                accs = list(accs)
                for u in range(2):
                    for c in range(CPR):
                        for k in range(NVEC):
                            accs[c * NVEC + k] = accs[c * NVEC + k] + rows_v[
                                p, c, 2 * l + u, pl.ds(k * NLANE, NLANE)
                            ]
                return tuple(accs)

            accs = lax.fori_loop(
                0,
                CHUNK // 2,
                acc_body,
                tuple(jnp.zeros((NLANE,), jnp.float32) for _ in range(CPR * NVEC)),
            )
            scale = jnp.float32(1.0 / L)
            for k in range(NVEC):
                tot = accs[k]
                for c in range(1, CPR):
                    tot = tot + accs[c * NVEC + k]
                acc_v[r, pl.ds(k * NLANE, NLANE)] = tot * scale

            nr = r + 4

            @pl.when(nr < ROWS_W)
            def _():
                fetch(nr, p)

        return 0

    lax.fori_loop(0, ROWS_W // 4, pair_body, 0)
    pltpu.sync_copy(acc_v, out_hbm.at[pl.ds(wid * ROWS_W, ROWS_W)])


def _sc_gather_mean(ids3, table):
    mesh = plsc.VectorSubcoreMesh(core_axis_name="c", subcore_axis_name="s")
    return pl.kernel(
        _sc_gather_mean_body,
        out_type=jax.ShapeDtypeStruct((B, D), jnp.float32),
        mesh=mesh,
        scratch_types=[
            pltpu.VMEM((NCH, CHUNK), jnp.int32),
            pltpu.VMEM((4, CPR, CHUNK, D), jnp.float32),
            pltpu.VMEM((ROWS_W, D), jnp.float32),
            pltpu.SemaphoreType.DMA,
            pltpu.SemaphoreType.DMA,
            pltpu.SemaphoreType.DMA,
            pltpu.SemaphoreType.DMA,
        ],
        compiler_params=pltpu.CompilerParams(use_tc_tiling_on_sc=False),
    )(ids3, table)


def _mlp_body(cur_ref, ctx_ref, w1a_ref, w1b_ref, b1_ref, w2_ref, b2_ref, out_ref):
    cur = cur_ref[...]
    ctx = ctx_ref[...]
    for i in range(NB):
        h = (
            jnp.dot(cur, w1a_ref[i], preferred_element_type=jnp.float32)
            + jnp.dot(ctx, w1b_ref[i], preferred_element_type=jnp.float32)
            + b1_ref[i][None, :]
        )
        h = jnp.maximum(h, 0.0)
        cur = cur + jnp.dot(h, w2_ref[i], preferred_element_type=jnp.float32) + b2_ref[i][None, :]
    out_ref[...] = cur


def _tc_mlp(cur0, ctx, W1, b1, W2, b2):
    w1a = W1[:, :D, :]
    w1b = W1[:, D:, :]
    bm = 512
    grid = B // bm
    return pl.pallas_call(
        _mlp_body,
        grid=(grid,),
        in_specs=[
            pl.BlockSpec((bm, D), lambda i: (i, 0)),
            pl.BlockSpec((bm, D), lambda i: (i, 0)),
            pl.BlockSpec((NB, D, H), lambda i: (0, 0, 0)),
            pl.BlockSpec((NB, D, H), lambda i: (0, 0, 0)),
            pl.BlockSpec((NB, H), lambda i: (0, 0)),
            pl.BlockSpec((NB, H, D), lambda i: (0, 0, 0)),
            pl.BlockSpec((NB, D), lambda i: (0, 0)),
        ],
        out_specs=pl.BlockSpec((bm, D), lambda i: (i, 0)),
        out_shape=jax.ShapeDtypeStruct((B, D), jnp.float32),
    )(cur0, ctx, w1a, w1b, b1, W2, b2)


def kernel(initial_noisy_embedding, context_ids, embedding_table, W1, b1, W2, b2):
    ids3 = context_ids.astype(jnp.int32).reshape(NW, NCH, CHUNK)
    table_lin = _tc_transpose(embedding_table.T).reshape(V, D)
    ctx = _sc_gather_mean(ids3, table_lin)
    return _tc_mlp(initial_noisy_embedding, ctx, W1, b1, W2, b2)
